# baseline (speedup 1.0000x reference)
import functools

import jax
import jax.numpy as jnp
from jax import lax
from jax.experimental import pallas as pl
from jax.experimental.pallas import tpu as pltpu
from jax.experimental.pallas import tpu_sc as plsc

N = 4000
E = 40000
D_IN = 256
H = 512
NC, NS = 2, 16
NW = NC * NS
CHUNK = 128
CPW = 10
EPAD = NW * CPW * CHUNK
NACC = 4096
RPT = NACC // NS
ZR = 64

_SRC_T = (0, 1, 0, 2, 1, 2)
_INC = ((1, 3), (0, 5), (2, 4))


SENT = -8


def _sc_aggregate(tab_all, glx_all, dlx_all, zrows, eye, np_, half_dst,
                  zsent, with_counts):
    mesh = plsc.VectorSubcoreMesh(core_axis_name="c", subcore_axis_name="s")
    EHALF = EPAD // NC
    CH = EHALF // CHUNK

    out_type = [jax.ShapeDtypeStruct((np_, NC, NACC, D_IN), jnp.float32)]
    if with_counts:
        out_type.append(jax.ShapeDtypeStruct((np_, NC, NACC), jnp.float32))

    @functools.partial(
        pl.kernel,
        out_type=out_type,
        mesh=mesh,
        scratch_types=[
            pltpu.VMEM((RPT, D_IN), jnp.float32),
            pltpu.VMEM((CHUNK, D_IN), jnp.float32),
            pltpu.VMEM((CHUNK,), jnp.int32),
            pltpu.VMEM((CHUNK,), jnp.int32),
            pltpu.VMEM((RPT,), jnp.float32),
            pltpu.VMEM((16, 16), jnp.float32),
            pltpu.SemaphoreType.DMA,
        ],
    )
    def k(*refs):
        if with_counts:
            tab, glx, dlx, zr, eyeh, out, cnt_out = refs[:7]
        else:
            tab, glx, dlx, zr, eyeh, out = refs[:6]
            cnt_out = None
        acc, rows, didx, glv, cntv, eyev, sem = refs[-7:]
        c = lax.axis_index("c")
        s = lax.axis_index("s")
        lo = s * RPT
        pltpu.sync_copy(eyeh, eyev)

        def phase_body(p, carry0):
            pd = p // 2 if half_dst else p
            for z in range(RPT // CHUNK):
                pltpu.sync_copy(zr, acc.at[pl.ds(z * CHUNK, CHUNK)])
            pltpu.sync_copy(zr.at[0], cntv)

            def chunk_body(j, carry):
                pltpu.sync_copy(dlx.at[pd, c, s, j], didx)
                pltpu.sync_copy(glx.at[p, c, s, j], glv)
                pltpu.async_copy(
                    tab.at[plsc.Indices(glv, ignored_value=zsent)],
                    rows, sem).wait()
                def group_body(kk, gcarry):
                    goff = pl.multiple_of(kk * 16, 16)
                    d16 = didx[pl.ds(goff, 16)]
                    for ln in range(16):
                        dls = d16[ln]

                        @pl.when(dls >= 0)
                        def _(dl=dls, r=goff + ln):
                            for kw in range(D_IN // 16):
                                plsc.addupdate(
                                    acc.at[dl, pl.ds(kw * 16, 16)],
                                    rows[r, pl.ds(kw * 16, 16)])
                            base = (dl // 16) * 16
                            plsc.addupdate(cntv.at[pl.ds(base, 16)],
                                           eyev[dl - base])
                    return gcarry

                lax.fori_loop(0, CHUNK // 16, group_body, 0)
                return carry

            lax.fori_loop(0, CH, chunk_body, 0)

            pltpu.sync_copy(acc, out.at[p, c, pl.ds(lo, RPT)])
            if with_counts:
                pltpu.sync_copy(cntv, cnt_out.at[p, c, pl.ds(lo, RPT)])
            return carry0

        lax.fori_loop(0, np_, phase_body, 0)

    return k(tab_all, glx_all, dlx_all, zrows, eye)


BM = 400
_DN = (((1,), (1,)), ((), ()))


def _tc1(P, cnt, a_et, b_et, xd, wal, wbl, war, wbr, ba, bb):
    def body(pa, pb, ct, x, al, bl, ar, br, b_a, b_b, o):
        ctv = ct[...]
        cnta = jnp.maximum(ctv[:, 2 * a_et:2 * a_et + 1]
                           + ctv[:, 2 * a_et + 1:2 * a_et + 2], 1.0)
        cntb = jnp.maximum(ctv[:, 2 * b_et:2 * b_et + 1]
                           + ctv[:, 2 * b_et + 1:2 * b_et + 2], 1.0)
        ma = (pa[0, 0] + pa[0, 1]) / cnta
        mb = (pb[0, 0] + pb[0, 1]) / cntb
        z = lax.dot_general(ma, al[...], _DN, preferred_element_type=jnp.float32)
        z = z + lax.dot_general(mb, bl[...], _DN, preferred_element_type=jnp.float32)
        z = z + lax.dot_general(x[...], ar[...] + br[...], _DN,
                                preferred_element_type=jnp.float32)
        o[...] = jnp.maximum(z + b_a[0] + b_b[0], 0.0)

    return pl.pallas_call(
        body,
        grid=(N // BM,),
        in_specs=[
            pl.BlockSpec((1, NC, BM, D_IN), lambda i, _a=a_et: (_a, 0, i, 0)),
            pl.BlockSpec((1, NC, BM, D_IN), lambda i, _b=b_et: (_b, 0, i, 0)),
            pl.BlockSpec((BM, 16), lambda i: (i, 0)),
            pl.BlockSpec((BM, D_IN), lambda i: (i, 0)),
            pl.BlockSpec((H, D_IN), lambda i: (0, 0)),
            pl.BlockSpec((H, D_IN), lambda i: (0, 0)),
            pl.BlockSpec((H, D_IN), lambda i: (0, 0)),
            pl.BlockSpec((H, D_IN), lambda i: (0, 0)),
            pl.BlockSpec((1, H), lambda i: (0, 0)),
            pl.BlockSpec((1, H), lambda i: (0, 0)),
        ],
        out_specs=pl.BlockSpec((BM, H), lambda i: (i, 0)),
        out_shape=jax.ShapeDtypeStruct((N, H), jnp.float32),
    )(P, P, cnt, xd, wal, wbl, war, wbr, ba, bb)


def _tc2(Q, cnt, a_et, b_et, hd, wal, wbl, war, wbr, b_a,
         b_b, wlin, blin):
    def body(a0, a1, b0, b1, ct, h, al, bl, ar, br, ba_, bb_, wl, bli, o):
        ctv = ct[...]
        cnta = jnp.maximum(ctv[:, 2 * a_et:2 * a_et + 1]
                           + ctv[:, 2 * a_et + 1:2 * a_et + 2], 1.0)
        cntb = jnp.maximum(ctv[:, 2 * b_et:2 * b_et + 1]
                           + ctv[:, 2 * b_et + 1:2 * b_et + 2], 1.0)
        ma0 = (a0[0, 0] + a0[0, 1]) / cnta
        ma1 = (a1[0, 0] + a1[0, 1]) / cnta
        mb0 = (b0[0, 0] + b0[0, 1]) / cntb
        mb1 = (b1[0, 0] + b1[0, 1]) / cntb
        alv, blv = al[...], bl[...]
        z = lax.dot_general(ma0, alv[:, :D_IN], _DN, preferred_element_type=jnp.float32)
        z = z + lax.dot_general(ma1, alv[:, D_IN:], _DN, preferred_element_type=jnp.float32)
        z = z + lax.dot_general(mb0, blv[:, :D_IN], _DN, preferred_element_type=jnp.float32)
        z = z + lax.dot_general(mb1, blv[:, D_IN:], _DN, preferred_element_type=jnp.float32)
        z = z + lax.dot_general(h[...], ar[...] + br[...], _DN,
                                preferred_element_type=jnp.float32)
        z = jnp.maximum(z + ba_[0] + bb_[0], 0.0)
        o[...] = lax.dot_general(z, wl[...], _DN,
                                 preferred_element_type=jnp.float32) + bli[0]

    return pl.pallas_call(
        body,
        grid=(N // BM,),
        in_specs=[
            pl.BlockSpec((1, NC, BM, D_IN), lambda i, _a=a_et: (2 * _a, 0, i, 0)),
            pl.BlockSpec((1, NC, BM, D_IN), lambda i, _a=a_et: (2 * _a + 1, 0, i, 0)),
            pl.BlockSpec((1, NC, BM, D_IN), lambda i, _b=b_et: (2 * _b, 0, i, 0)),
            pl.BlockSpec((1, NC, BM, D_IN), lambda i, _b=b_et: (2 * _b + 1, 0, i, 0)),
            pl.BlockSpec((BM, 16), lambda i: (i, 0)),
            pl.BlockSpec((BM, H), lambda i: (i, 0)),
            pl.BlockSpec((H, H), lambda i: (0, 0)),
            pl.BlockSpec((H, H), lambda i: (0, 0)),
            pl.BlockSpec((H, H), lambda i: (0, 0)),
            pl.BlockSpec((H, H), lambda i: (0, 0)),
            pl.BlockSpec((1, H), lambda i: (0, 0)),
            pl.BlockSpec((1, H), lambda i: (0, 0)),
            pl.BlockSpec((H, H), lambda i: (0, 0)),
            pl.BlockSpec((1, H), lambda i: (0, 0)),
        ],
        out_specs=pl.BlockSpec((BM, H), lambda i: (i, 0)),
        out_shape=jax.ShapeDtypeStruct((N, H), jnp.float32),
    )(Q, Q, Q, Q, cnt, hd, wal, wbl, war, wbr, b_a, b_b, wlin,
      blin)


def kernel(x_disease, x_drug, x_gene,
           ei_dis_drug, ei_drug_dis, ei_dis_gene, ei_gene_dis, ei_drug_gene,
           ei_gene_drug,
           W1_0_l, W1_0_r, b1_0, W2_0_l, W2_0_r, b2_0,
           W1_1_l, W1_1_r, b1_1, W2_1_l, W2_1_r, b2_1,
           W1_2_l, W1_2_r, b1_2, W2_2_l, W2_2_r, b2_2,
           W1_3_l, W1_3_r, b1_3, W2_3_l, W2_3_r, b2_3,
           W1_4_l, W1_4_r, b1_4, W2_4_l, W2_4_r, b2_4,
           W1_5_l, W1_5_r, b1_5, W2_5_l, W2_5_r, b2_5,
           Wlin_disease, blin_disease, Wlin_drug, blin_drug, Wlin_gene,
           blin_gene):
    f32 = jnp.float32
    xs = (x_disease, x_drug, x_gene)
    eis = (ei_dis_drug, ei_drug_dis, ei_dis_gene, ei_gene_dis, ei_drug_gene,
           ei_gene_drug)
    W1l = (W1_0_l, W1_1_l, W1_2_l, W1_3_l, W1_4_l, W1_5_l)
    W1r = (W1_0_r, W1_1_r, W1_2_r, W1_3_r, W1_4_r, W1_5_r)
    b1 = (b1_0, b1_1, b1_2, b1_3, b1_4, b1_5)
    W2l = (W2_0_l, W2_1_l, W2_2_l, W2_3_l, W2_4_l, W2_5_l)
    W2r = (W2_0_r, W2_1_r, W2_2_r, W2_3_r, W2_4_r, W2_5_r)
    b2 = (b2_0, b2_1, b2_2, b2_3, b2_4, b2_5)
    Wlin = (Wlin_disease, Wlin_drug, Wlin_gene)
    blin = (blin_disease, blin_drug, blin_gene)

    pad = EPAD - E
    CH = EPAD // NC // CHUNK
    sids = jnp.arange(NS, dtype=jnp.int32).reshape(1, NS, 1, 1)

    def glx_of(srcp, dstp, zbase):
        s4 = srcp.reshape(NC, 1, CH, CHUNK)
        d4 = dstp.reshape(NC, 1, CH, CHUNK)
        return jnp.where(d4 // RPT == sids, s4, zbase)

    def dlx_of(dstp):
        d4 = dstp.reshape(NC, 1, CH, CHUNK)
        return jnp.where(d4 // RPT == sids, d4 - sids * RPT, -1)

    T1R = N + 8
    T2R = 2 * N + 8
    glx1, dlxs, glx2 = [], [], []
    for et, e in enumerate(eis):
        s_, d_ = e[0], e[1]
        dstp = jnp.pad(d_, (0, pad), constant_values=N)
        dlxs.append(dlx_of(dstp))
        off1 = _SRC_T[et] * T1R
        off2 = _SRC_T[et] * T2R
        srcp = jnp.pad(s_, (0, pad), constant_values=N) + off1
        glx1.append(glx_of(srcp, dstp, 3 * T1R))
        s2 = s_ * 2
        glx2.append(glx_of(
            jnp.pad(s2, (0, pad), constant_values=2 * N) + off2, dstp, 3 * T2R))
        glx2.append(glx_of(
            jnp.pad(s2 + 1, (0, pad), constant_values=2 * N) + off2, dstp,
            3 * T2R))
    glx1 = jnp.stack(glx1)
    glx2 = jnp.stack(glx2)
    dlxs = jnp.stack(dlxs)

    zpad = jnp.zeros((8, D_IN), f32)
    zblock = jnp.zeros((CHUNK, D_IN), f32)
    t1 = jnp.concatenate(
        [jnp.concatenate([x, zpad], 0) for x in xs] + [zblock], 0)
    zr = jnp.zeros((CHUNK, D_IN), f32)
    eye = jnp.eye(16, dtype=f32)
    P, cnt_raw = _sc_aggregate(t1, glx1, dlxs, zr, eye, 6, False,
                               3 * T1R, with_counts=True)
    cnt = jnp.pad(jnp.transpose(cnt_raw, (2, 0, 1)).reshape(NACC, 12),
                  ((0, 0), (0, 4)))

    h = []
    for d in range(3):
        a, b = _INC[d]
        h.append(_tc1(P, cnt, a, b, xs[d], W1l[a], W1l[b], W1r[a],
                      W1r[b], b1[a].reshape(1, H), b1[b].reshape(1, H)))

    t2 = jnp.concatenate(
        [jnp.concatenate([hh.reshape(2 * N, D_IN), zpad], 0) for hh in h]
        + [zblock], 0)
    (Q,) = _sc_aggregate(t2, glx2, dlxs, zr, eye, 12, True,
                         3 * T2R, with_counts=False)

    outs = []
    for d in range(3):
        a, b = _INC[d]
        outs.append(_tc2(Q, cnt, a, b, h[d],
                         W2l[a], W2l[b], W2r[a], W2r[b],
                         b2[a].reshape(1, H), b2[b].reshape(1, H),
                         Wlin[d], blin[d].reshape(1, H)))
    feat = jnp.concatenate(outs, 0)
    return (outs[0], outs[1], outs[2], feat)


# speedup vs baseline: 13.7697x; 13.7697x over previous
import functools

import jax
import jax.numpy as jnp
from jax import lax
from jax.experimental import pallas as pl
from jax.experimental.pallas import tpu as pltpu
from jax.experimental.pallas import tpu_sc as plsc

N = 4000
E = 40000
D_IN = 256
H = 512
NC, NS = 2, 16
NW = NC * NS
CHUNK = 128
CPW = 10
EPAD = NW * CPW * CHUNK
NACC = 4096
RPT = NACC // NS
ZR = 64

_SRC_T = (0, 1, 0, 2, 1, 2)
_INC = ((1, 3), (0, 5), (2, 4))


SENT = -8


def _sc_aggregate(tab_all, glx_all, dlx_all, zrows, eye, np_, half_dst,
                  zsent, with_counts):
    mesh = plsc.VectorSubcoreMesh(core_axis_name="c", subcore_axis_name="s")
    EHALF = EPAD // NC
    CH = EHALF // CHUNK

    out_type = [jax.ShapeDtypeStruct((np_, NC, NACC, D_IN), jnp.float32)]
    if with_counts:
        out_type.append(jax.ShapeDtypeStruct((np_, NC, NACC), jnp.float32))

    @functools.partial(
        pl.kernel,
        out_type=out_type,
        mesh=mesh,
        scratch_types=[
            pltpu.VMEM((RPT, D_IN), jnp.float32),
            pltpu.VMEM((CHUNK, D_IN), jnp.float32),
            pltpu.VMEM((CHUNK,), jnp.int32),
            pltpu.VMEM((CHUNK,), jnp.int32),
            pltpu.VMEM((RPT,), jnp.float32),
            pltpu.VMEM((16, 16), jnp.float32),
            pltpu.SemaphoreType.DMA,
        ],
    )
    def k(*refs):
        if with_counts:
            tab, glx, dlx, zr, eyeh, out, cnt_out = refs[:7]
        else:
            tab, glx, dlx, zr, eyeh, out = refs[:6]
            cnt_out = None
        acc, rows, didx, glv, cntv, eyev, sem = refs[-7:]
        c = lax.axis_index("c")
        s = lax.axis_index("s")
        lo = s * RPT
        pltpu.sync_copy(eyeh, eyev)

        def phase_body(p, carry0):
            pd = p // 2 if half_dst else p
            for z in range(RPT // CHUNK):
                pltpu.sync_copy(zr, acc.at[pl.ds(z * CHUNK, CHUNK)])
            pltpu.sync_copy(zr.at[0], cntv)

            def chunk_body(j, carry):
                pltpu.sync_copy(dlx.at[pd, c, s, j], didx)
                pltpu.sync_copy(glx.at[p, c, s, j], glv)
                pltpu.async_copy(tab.at[glv], rows, sem).wait()
                def group_body(kk, gcarry):
                    goff = pl.multiple_of(kk * 16, 16)
                    d16 = didx[pl.ds(goff, 16)]
                    for ln in range(16):
                        dls = d16[ln]

                        @pl.when(dls >= 0)
                        def _(dl=dls, r=goff + ln):
                            for kw in range(D_IN // 16):
                                plsc.addupdate(
                                    acc.at[dl, pl.ds(kw * 16, 16)],
                                    rows[r, pl.ds(kw * 16, 16)])
                            base = (dl // 16) * 16
                            plsc.addupdate(cntv.at[pl.ds(base, 16)],
                                           eyev[dl - base])
                    return gcarry

                lax.fori_loop(0, CHUNK // 16, group_body, 0)
                return carry

            lax.fori_loop(0, CH, chunk_body, 0)

            pltpu.sync_copy(acc, out.at[p, c, pl.ds(lo, RPT)])
            if with_counts:
                pltpu.sync_copy(cntv, cnt_out.at[p, c, pl.ds(lo, RPT)])
            return carry0

        lax.fori_loop(0, np_, phase_body, 0)

    return k(tab_all, glx_all, dlx_all, zrows, eye)


BM = 400
_DN = (((1,), (1,)), ((), ()))


def _tc1(P, cnt, a_et, b_et, xd, wal, wbl, war, wbr, ba, bb):
    def body(pa, pb, ct, x, al, bl, ar, br, b_a, b_b, o):
        ctv = ct[...]
        cnta = jnp.maximum(ctv[:, 2 * a_et:2 * a_et + 1]
                           + ctv[:, 2 * a_et + 1:2 * a_et + 2], 1.0)
        cntb = jnp.maximum(ctv[:, 2 * b_et:2 * b_et + 1]
                           + ctv[:, 2 * b_et + 1:2 * b_et + 2], 1.0)
        ma = (pa[0, 0] + pa[0, 1]) / cnta
        mb = (pb[0, 0] + pb[0, 1]) / cntb
        z = lax.dot_general(ma, al[...], _DN, preferred_element_type=jnp.float32)
        z = z + lax.dot_general(mb, bl[...], _DN, preferred_element_type=jnp.float32)
        z = z + lax.dot_general(x[...], ar[...] + br[...], _DN,
                                preferred_element_type=jnp.float32)
        o[...] = jnp.maximum(z + b_a[0] + b_b[0], 0.0)

    return pl.pallas_call(
        body,
        grid=(N // BM,),
        in_specs=[
            pl.BlockSpec((1, NC, BM, D_IN), lambda i, _a=a_et: (_a, 0, i, 0)),
            pl.BlockSpec((1, NC, BM, D_IN), lambda i, _b=b_et: (_b, 0, i, 0)),
            pl.BlockSpec((BM, 16), lambda i: (i, 0)),
            pl.BlockSpec((BM, D_IN), lambda i: (i, 0)),
            pl.BlockSpec((H, D_IN), lambda i: (0, 0)),
            pl.BlockSpec((H, D_IN), lambda i: (0, 0)),
            pl.BlockSpec((H, D_IN), lambda i: (0, 0)),
            pl.BlockSpec((H, D_IN), lambda i: (0, 0)),
            pl.BlockSpec((1, H), lambda i: (0, 0)),
            pl.BlockSpec((1, H), lambda i: (0, 0)),
        ],
        out_specs=pl.BlockSpec((BM, H), lambda i: (i, 0)),
        out_shape=jax.ShapeDtypeStruct((N, H), jnp.float32),
    )(P, P, cnt, xd, wal, wbl, war, wbr, ba, bb)


def _tc2(Q, cnt, a_et, b_et, hd, wal, wbl, war, wbr, b_a,
         b_b, wlin, blin):
    def body(a0, a1, b0, b1, ct, h, al, bl, ar, br, ba_, bb_, wl, bli, o):
        ctv = ct[...]
        cnta = jnp.maximum(ctv[:, 2 * a_et:2 * a_et + 1]
                           + ctv[:, 2 * a_et + 1:2 * a_et + 2], 1.0)
        cntb = jnp.maximum(ctv[:, 2 * b_et:2 * b_et + 1]
                           + ctv[:, 2 * b_et + 1:2 * b_et + 2], 1.0)
        ma0 = (a0[0, 0] + a0[0, 1]) / cnta
        ma1 = (a1[0, 0] + a1[0, 1]) / cnta
        mb0 = (b0[0, 0] + b0[0, 1]) / cntb
        mb1 = (b1[0, 0] + b1[0, 1]) / cntb
        alv, blv = al[...], bl[...]
        z = lax.dot_general(ma0, alv[:, :D_IN], _DN, preferred_element_type=jnp.float32)
        z = z + lax.dot_general(ma1, alv[:, D_IN:], _DN, preferred_element_type=jnp.float32)
        z = z + lax.dot_general(mb0, blv[:, :D_IN], _DN, preferred_element_type=jnp.float32)
        z = z + lax.dot_general(mb1, blv[:, D_IN:], _DN, preferred_element_type=jnp.float32)
        z = z + lax.dot_general(h[...], ar[...] + br[...], _DN,
                                preferred_element_type=jnp.float32)
        z = jnp.maximum(z + ba_[0] + bb_[0], 0.0)
        o[...] = lax.dot_general(z, wl[...], _DN,
                                 preferred_element_type=jnp.float32) + bli[0]

    return pl.pallas_call(
        body,
        grid=(N // BM,),
        in_specs=[
            pl.BlockSpec((1, NC, BM, D_IN), lambda i, _a=a_et: (2 * _a, 0, i, 0)),
            pl.BlockSpec((1, NC, BM, D_IN), lambda i, _a=a_et: (2 * _a + 1, 0, i, 0)),
            pl.BlockSpec((1, NC, BM, D_IN), lambda i, _b=b_et: (2 * _b, 0, i, 0)),
            pl.BlockSpec((1, NC, BM, D_IN), lambda i, _b=b_et: (2 * _b + 1, 0, i, 0)),
            pl.BlockSpec((BM, 16), lambda i: (i, 0)),
            pl.BlockSpec((BM, H), lambda i: (i, 0)),
            pl.BlockSpec((H, H), lambda i: (0, 0)),
            pl.BlockSpec((H, H), lambda i: (0, 0)),
            pl.BlockSpec((H, H), lambda i: (0, 0)),
            pl.BlockSpec((H, H), lambda i: (0, 0)),
            pl.BlockSpec((1, H), lambda i: (0, 0)),
            pl.BlockSpec((1, H), lambda i: (0, 0)),
            pl.BlockSpec((H, H), lambda i: (0, 0)),
            pl.BlockSpec((1, H), lambda i: (0, 0)),
        ],
        out_specs=pl.BlockSpec((BM, H), lambda i: (i, 0)),
        out_shape=jax.ShapeDtypeStruct((N, H), jnp.float32),
    )(Q, Q, Q, Q, cnt, hd, wal, wbl, war, wbr, b_a, b_b, wlin,
      blin)


def kernel(x_disease, x_drug, x_gene,
           ei_dis_drug, ei_drug_dis, ei_dis_gene, ei_gene_dis, ei_drug_gene,
           ei_gene_drug,
           W1_0_l, W1_0_r, b1_0, W2_0_l, W2_0_r, b2_0,
           W1_1_l, W1_1_r, b1_1, W2_1_l, W2_1_r, b2_1,
           W1_2_l, W1_2_r, b1_2, W2_2_l, W2_2_r, b2_2,
           W1_3_l, W1_3_r, b1_3, W2_3_l, W2_3_r, b2_3,
           W1_4_l, W1_4_r, b1_4, W2_4_l, W2_4_r, b2_4,
           W1_5_l, W1_5_r, b1_5, W2_5_l, W2_5_r, b2_5,
           Wlin_disease, blin_disease, Wlin_drug, blin_drug, Wlin_gene,
           blin_gene):
    f32 = jnp.float32
    xs = (x_disease, x_drug, x_gene)
    eis = (ei_dis_drug, ei_drug_dis, ei_dis_gene, ei_gene_dis, ei_drug_gene,
           ei_gene_drug)
    W1l = (W1_0_l, W1_1_l, W1_2_l, W1_3_l, W1_4_l, W1_5_l)
    W1r = (W1_0_r, W1_1_r, W1_2_r, W1_3_r, W1_4_r, W1_5_r)
    b1 = (b1_0, b1_1, b1_2, b1_3, b1_4, b1_5)
    W2l = (W2_0_l, W2_1_l, W2_2_l, W2_3_l, W2_4_l, W2_5_l)
    W2r = (W2_0_r, W2_1_r, W2_2_r, W2_3_r, W2_4_r, W2_5_r)
    b2 = (b2_0, b2_1, b2_2, b2_3, b2_4, b2_5)
    Wlin = (Wlin_disease, Wlin_drug, Wlin_gene)
    blin = (blin_disease, blin_drug, blin_gene)

    pad = EPAD - E
    CH = EPAD // NC // CHUNK
    sids = jnp.arange(NS, dtype=jnp.int32).reshape(1, NS, 1, 1)

    lanepos = (jnp.arange(EPAD, dtype=jnp.int32) % CHUNK).reshape(
        NC, 1, EPAD // NC // CHUNK, CHUNK)

    def glx_of(srcp, dstp, zbase):
        s4 = srcp.reshape(NC, 1, CH, CHUNK)
        d4 = dstp.reshape(NC, 1, CH, CHUNK)
        return jnp.where(d4 // RPT == sids, s4, zbase + lanepos)

    def dlx_of(dstp):
        d4 = dstp.reshape(NC, 1, CH, CHUNK)
        return jnp.where(d4 // RPT == sids, d4 - sids * RPT, -1)

    T1R = N + 8
    T2R = 2 * N + 8
    glx1, dlxs, glx2 = [], [], []
    for et, e in enumerate(eis):
        s_, d_ = e[0], e[1]
        dstp = jnp.pad(d_, (0, pad), constant_values=N)
        dlxs.append(dlx_of(dstp))
        off1 = _SRC_T[et] * T1R
        off2 = _SRC_T[et] * T2R
        srcp = jnp.pad(s_, (0, pad), constant_values=N) + off1
        glx1.append(glx_of(srcp, dstp, 3 * T1R))
        s2 = s_ * 2
        glx2.append(glx_of(
            jnp.pad(s2, (0, pad), constant_values=2 * N) + off2, dstp, 3 * T2R))
        glx2.append(glx_of(
            jnp.pad(s2 + 1, (0, pad), constant_values=2 * N) + off2, dstp,
            3 * T2R))
    glx1 = jnp.stack(glx1)
    glx2 = jnp.stack(glx2)
    dlxs = jnp.stack(dlxs)

    zpad = jnp.zeros((8, D_IN), f32)
    zblock = jnp.zeros((CHUNK, D_IN), f32)
    t1 = jnp.concatenate(
        [jnp.concatenate([x, zpad], 0) for x in xs] + [zblock], 0)
    zr = jnp.zeros((CHUNK, D_IN), f32)
    eye = jnp.eye(16, dtype=f32)
    P, cnt_raw = _sc_aggregate(t1, glx1, dlxs, zr, eye, 6, False,
                               3 * T1R, with_counts=True)
    cnt = jnp.pad(jnp.transpose(cnt_raw, (2, 0, 1)).reshape(NACC, 12),
                  ((0, 0), (0, 4)))

    h = []
    for d in range(3):
        a, b = _INC[d]
        h.append(_tc1(P, cnt, a, b, xs[d], W1l[a], W1l[b], W1r[a],
                      W1r[b], b1[a].reshape(1, H), b1[b].reshape(1, H)))

    t2 = jnp.concatenate(
        [jnp.concatenate([hh.reshape(2 * N, D_IN), zpad], 0) for hh in h]
        + [zblock], 0)
    (Q,) = _sc_aggregate(t2, glx2, dlxs, zr, eye, 12, True,
                         3 * T2R, with_counts=False)

    outs = []
    for d in range(3):
        a, b = _INC[d]
        outs.append(_tc2(Q, cnt, a, b, h[d],
                         W2l[a], W2l[b], W2r[a], W2r[b],
                         b2[a].reshape(1, H), b2[b].reshape(1, H),
                         Wlin[d], blin[d].reshape(1, H)))
    feat = jnp.concatenate(outs, 0)
    return (outs[0], outs[1], outs[2], feat)


# speedup vs baseline: 25.5951x; 1.8588x over previous
import functools

import jax
import jax.numpy as jnp
from jax import lax
from jax.experimental import pallas as pl
from jax.experimental.pallas import tpu as pltpu
from jax.experimental.pallas import tpu_sc as plsc

N = 4000
E = 40000
D_IN = 256
H = 512
NC, NS = 2, 16
NW = NC * NS
CHUNK = 128
CPW = 10
EPAD = NW * CPW * CHUNK
NACC = 4096
RPT = NACC // NS
ZR = 64

_SRC_T = (0, 1, 0, 2, 1, 2)
_INC = ((1, 3), (0, 5), (2, 4))


SENT = -8


def _sc_aggregate(tab_all, glx_all, dst_all, zrows, eye, np_, half_dst,
                  with_counts):
    mesh = plsc.VectorSubcoreMesh(core_axis_name="c", subcore_axis_name="s")
    EHALF = EPAD // NC
    CH = EHALF // CHUNK

    out_type = [jax.ShapeDtypeStruct((np_, NC, NACC, D_IN), jnp.float32)]
    if with_counts:
        out_type.append(jax.ShapeDtypeStruct((np_, NC, NACC), jnp.float32))

    @functools.partial(
        pl.kernel,
        out_type=out_type,
        mesh=mesh,
        scratch_types=[
            pltpu.VMEM((RPT, D_IN), jnp.float32),
            pltpu.VMEM((CHUNK, D_IN), jnp.float32),
            pltpu.VMEM((CHUNK,), jnp.int32),
            pltpu.VMEM((CHUNK,), jnp.int32),
            pltpu.VMEM((RPT,), jnp.float32),
            pltpu.VMEM((16, 16), jnp.float32),
            pltpu.SemaphoreType.DMA,
        ],
    )
    def k(*refs):
        if with_counts:
            tab, glx, dst, zr, eyeh, out, cnt_out = refs[:7]
        else:
            tab, glx, dst, zr, eyeh, out = refs[:6]
            cnt_out = None
        acc, rows, didx, glv, cntv, eyev, sem = refs[-7:]
        c = lax.axis_index("c")
        s = lax.axis_index("s")
        lo = s * RPT
        pltpu.sync_copy(eyeh, eyev)

        def phase_body(p, carry0):
            pd = p // 2 if half_dst else p
            for z in range(RPT // CHUNK):
                pltpu.sync_copy(zr, acc.at[pl.ds(z * CHUNK, CHUNK)])
            pltpu.sync_copy(zr.at[0], cntv)

            def chunk_body(j, carry):
                pltpu.sync_copy(dst.at[pd, c * CH + j], didx)
                pltpu.sync_copy(glx.at[p, c, s, j], glv)
                pltpu.async_copy(tab.at[glv], rows, sem).wait()
                def group_body(kk, gcarry):
                    goff = pl.multiple_of(kk * 16, 16)
                    d16 = didx[pl.ds(goff, 16)]
                    for ln in range(16):
                        dls = d16[ln]

                        @pl.when((dls >= lo) & (dls < lo + RPT))
                        def _(dl=dls - lo, r=goff + ln):
                            for kw in range(D_IN // 16):
                                plsc.addupdate(
                                    acc.at[dl, pl.ds(kw * 16, 16)],
                                    rows[r, pl.ds(kw * 16, 16)])
                            base = (dl // 16) * 16
                            plsc.addupdate(cntv.at[pl.ds(base, 16)],
                                           eyev[dl - base])
                    return gcarry

                lax.fori_loop(0, CHUNK // 16, group_body, 0)
                return carry

            lax.fori_loop(0, CH, chunk_body, 0)

            pltpu.sync_copy(acc, out.at[p, c, pl.ds(lo, RPT)])
            if with_counts:
                pltpu.sync_copy(cntv, cnt_out.at[p, c, pl.ds(lo, RPT)])
            return carry0

        lax.fori_loop(0, np_, phase_body, 0)

    return k(tab_all, glx_all, dst_all, zrows, eye)


BM = 400
_DN = (((1,), (1,)), ((), ()))


def _tc1(P, cnt, a_et, b_et, xd, wal, wbl, war, wbr, ba, bb):
    def body(pa, pb, ct, x, al, bl, ar, br, b_a, b_b, o):
        ctv = ct[...]
        cnta = jnp.maximum(ctv[:, 2 * a_et:2 * a_et + 1]
                           + ctv[:, 2 * a_et + 1:2 * a_et + 2], 1.0)
        cntb = jnp.maximum(ctv[:, 2 * b_et:2 * b_et + 1]
                           + ctv[:, 2 * b_et + 1:2 * b_et + 2], 1.0)
        ma = (pa[0, 0] + pa[0, 1]) / cnta
        mb = (pb[0, 0] + pb[0, 1]) / cntb
        z = lax.dot_general(ma, al[...], _DN, preferred_element_type=jnp.float32)
        z = z + lax.dot_general(mb, bl[...], _DN, preferred_element_type=jnp.float32)
        z = z + lax.dot_general(x[...], ar[...] + br[...], _DN,
                                preferred_element_type=jnp.float32)
        o[...] = jnp.maximum(z + b_a[0] + b_b[0], 0.0)

    return pl.pallas_call(
        body,
        grid=(N // BM,),
        in_specs=[
            pl.BlockSpec((1, NC, BM, D_IN), lambda i, _a=a_et: (_a, 0, i, 0)),
            pl.BlockSpec((1, NC, BM, D_IN), lambda i, _b=b_et: (_b, 0, i, 0)),
            pl.BlockSpec((BM, 16), lambda i: (i, 0)),
            pl.BlockSpec((BM, D_IN), lambda i: (i, 0)),
            pl.BlockSpec((H, D_IN), lambda i: (0, 0)),
            pl.BlockSpec((H, D_IN), lambda i: (0, 0)),
            pl.BlockSpec((H, D_IN), lambda i: (0, 0)),
            pl.BlockSpec((H, D_IN), lambda i: (0, 0)),
            pl.BlockSpec((1, H), lambda i: (0, 0)),
            pl.BlockSpec((1, H), lambda i: (0, 0)),
        ],
        out_specs=pl.BlockSpec((BM, H), lambda i: (i, 0)),
        out_shape=jax.ShapeDtypeStruct((N, H), jnp.float32),
    )(P, P, cnt, xd, wal, wbl, war, wbr, ba, bb)


def _tc2(Q, cnt, a_et, b_et, hd, wal, wbl, war, wbr, b_a,
         b_b, wlin, blin):
    def body(a0, a1, b0, b1, ct, h, al, bl, ar, br, ba_, bb_, wl, bli, o):
        ctv = ct[...]
        cnta = jnp.maximum(ctv[:, 2 * a_et:2 * a_et + 1]
                           + ctv[:, 2 * a_et + 1:2 * a_et + 2], 1.0)
        cntb = jnp.maximum(ctv[:, 2 * b_et:2 * b_et + 1]
                           + ctv[:, 2 * b_et + 1:2 * b_et + 2], 1.0)
        ma0 = (a0[0, 0] + a0[0, 1]) / cnta
        ma1 = (a1[0, 0] + a1[0, 1]) / cnta
        mb0 = (b0[0, 0] + b0[0, 1]) / cntb
        mb1 = (b1[0, 0] + b1[0, 1]) / cntb
        alv, blv = al[...], bl[...]
        z = lax.dot_general(ma0, alv[:, :D_IN], _DN, preferred_element_type=jnp.float32)
        z = z + lax.dot_general(ma1, alv[:, D_IN:], _DN, preferred_element_type=jnp.float32)
        z = z + lax.dot_general(mb0, blv[:, :D_IN], _DN, preferred_element_type=jnp.float32)
        z = z + lax.dot_general(mb1, blv[:, D_IN:], _DN, preferred_element_type=jnp.float32)
        z = z + lax.dot_general(h[...], ar[...] + br[...], _DN,
                                preferred_element_type=jnp.float32)
        z = jnp.maximum(z + ba_[0] + bb_[0], 0.0)
        o[...] = lax.dot_general(z, wl[...], _DN,
                                 preferred_element_type=jnp.float32) + bli[0]

    return pl.pallas_call(
        body,
        grid=(N // BM,),
        in_specs=[
            pl.BlockSpec((1, NC, BM, D_IN), lambda i, _a=a_et: (2 * _a, 0, i, 0)),
            pl.BlockSpec((1, NC, BM, D_IN), lambda i, _a=a_et: (2 * _a + 1, 0, i, 0)),
            pl.BlockSpec((1, NC, BM, D_IN), lambda i, _b=b_et: (2 * _b, 0, i, 0)),
            pl.BlockSpec((1, NC, BM, D_IN), lambda i, _b=b_et: (2 * _b + 1, 0, i, 0)),
            pl.BlockSpec((BM, 16), lambda i: (i, 0)),
            pl.BlockSpec((BM, H), lambda i: (i, 0)),
            pl.BlockSpec((H, H), lambda i: (0, 0)),
            pl.BlockSpec((H, H), lambda i: (0, 0)),
            pl.BlockSpec((H, H), lambda i: (0, 0)),
            pl.BlockSpec((H, H), lambda i: (0, 0)),
            pl.BlockSpec((1, H), lambda i: (0, 0)),
            pl.BlockSpec((1, H), lambda i: (0, 0)),
            pl.BlockSpec((H, H), lambda i: (0, 0)),
            pl.BlockSpec((1, H), lambda i: (0, 0)),
        ],
        out_specs=pl.BlockSpec((BM, H), lambda i: (i, 0)),
        out_shape=jax.ShapeDtypeStruct((N, H), jnp.float32),
    )(Q, Q, Q, Q, cnt, hd, wal, wbl, war, wbr, b_a, b_b, wlin,
      blin)


def kernel(x_disease, x_drug, x_gene,
           ei_dis_drug, ei_drug_dis, ei_dis_gene, ei_gene_dis, ei_drug_gene,
           ei_gene_drug,
           W1_0_l, W1_0_r, b1_0, W2_0_l, W2_0_r, b2_0,
           W1_1_l, W1_1_r, b1_1, W2_1_l, W2_1_r, b2_1,
           W1_2_l, W1_2_r, b1_2, W2_2_l, W2_2_r, b2_2,
           W1_3_l, W1_3_r, b1_3, W2_3_l, W2_3_r, b2_3,
           W1_4_l, W1_4_r, b1_4, W2_4_l, W2_4_r, b2_4,
           W1_5_l, W1_5_r, b1_5, W2_5_l, W2_5_r, b2_5,
           Wlin_disease, blin_disease, Wlin_drug, blin_drug, Wlin_gene,
           blin_gene):
    f32 = jnp.float32
    xs = (x_disease, x_drug, x_gene)
    eis = (ei_dis_drug, ei_drug_dis, ei_dis_gene, ei_gene_dis, ei_drug_gene,
           ei_gene_drug)
    W1l = (W1_0_l, W1_1_l, W1_2_l, W1_3_l, W1_4_l, W1_5_l)
    W1r = (W1_0_r, W1_1_r, W1_2_r, W1_3_r, W1_4_r, W1_5_r)
    b1 = (b1_0, b1_1, b1_2, b1_3, b1_4, b1_5)
    W2l = (W2_0_l, W2_1_l, W2_2_l, W2_3_l, W2_4_l, W2_5_l)
    W2r = (W2_0_r, W2_1_r, W2_2_r, W2_3_r, W2_4_r, W2_5_r)
    b2 = (b2_0, b2_1, b2_2, b2_3, b2_4, b2_5)
    Wlin = (Wlin_disease, Wlin_drug, Wlin_gene)
    blin = (blin_disease, blin_drug, blin_gene)

    pad = EPAD - E
    CH = EPAD // NC // CHUNK
    sids = jnp.arange(NS, dtype=jnp.int32).reshape(1, NS, 1, 1)

    lanepos = (jnp.arange(EPAD, dtype=jnp.int32) % CHUNK).reshape(
        NC, 1, EPAD // NC // CHUNK, CHUNK)

    def glx_of(srcp, dstp, zbase):
        s4 = srcp.reshape(NC, 1, CH, CHUNK)
        d4 = dstp.reshape(NC, 1, CH, CHUNK)
        return jnp.where(d4 // RPT == sids, s4, zbase + lanepos)

    T1R = N + 8
    T2R = 2 * N + 8
    glx1, dsts, glx2 = [], [], []
    for et, e in enumerate(eis):
        s_, d_ = e[0], e[1]
        dstp = jnp.pad(d_, (0, pad), constant_values=N)
        dsts.append(dstp.reshape(NW * CPW, CHUNK))
        off1 = _SRC_T[et] * T1R
        off2 = _SRC_T[et] * T2R
        srcp = jnp.pad(s_, (0, pad), constant_values=N) + off1
        glx1.append(glx_of(srcp, dstp, 3 * T1R))
        s2 = s_ * 2
        glx2.append(glx_of(
            jnp.pad(s2, (0, pad), constant_values=2 * N) + off2, dstp, 3 * T2R))
        glx2.append(glx_of(
            jnp.pad(s2 + 1, (0, pad), constant_values=2 * N) + off2, dstp,
            3 * T2R))
    glx1 = jnp.stack(glx1)
    glx2 = jnp.stack(glx2)
    dsts = jnp.stack(dsts)

    zpad = jnp.zeros((8, D_IN), f32)
    zblock = jnp.zeros((CHUNK, D_IN), f32)
    t1 = jnp.concatenate(
        [jnp.concatenate([x, zpad], 0) for x in xs] + [zblock], 0)
    zr = jnp.zeros((CHUNK, D_IN), f32)
    eye = jnp.eye(16, dtype=f32)
    P, cnt_raw = _sc_aggregate(t1, glx1, dsts, zr, eye, 6, False,
                               with_counts=True)
    cnt = jnp.pad(jnp.transpose(cnt_raw, (2, 0, 1)).reshape(NACC, 12),
                  ((0, 0), (0, 4)))

    h = []
    for d in range(3):
        a, b = _INC[d]
        h.append(_tc1(P, cnt, a, b, xs[d], W1l[a], W1l[b], W1r[a],
                      W1r[b], b1[a].reshape(1, H), b1[b].reshape(1, H)))

    t2 = jnp.concatenate(
        [jnp.concatenate([hh.reshape(2 * N, D_IN), zpad], 0) for hh in h]
        + [zblock], 0)
    (Q,) = _sc_aggregate(t2, glx2, dsts, zr, eye, 12, True,
                         with_counts=False)

    outs = []
    for d in range(3):
        a, b = _INC[d]
        outs.append(_tc2(Q, cnt, a, b, h[d],
                         W2l[a], W2l[b], W2r[a], W2r[b],
                         b2[a].reshape(1, H), b2[b].reshape(1, H),
                         Wlin[d], blin[d].reshape(1, H)))
    feat = jnp.concatenate(outs, 0)
    return (outs[0], outs[1], outs[2], feat)
